# baseline (speedup 1.0000x reference)
import functools

import jax
import jax.numpy as jnp
from jax import lax
from jax.experimental import pallas as pl
from jax.experimental.pallas import tpu as pltpu
from jax.experimental.pallas import tpu_sc as plsc

_N = 10000
_D = 128
_K = 32

_NW = 32
_EPW = 10240
_AEB = 128
_ANB = _EPW // _AEB
_AQ = 5
_AQNB = _ANB // _AQ
_NBUF = 2
_DEB = 64
_DNB = _EPW // _DEB
_ROWS_PER_TILE = 632
_NPAD = _ROWS_PER_TILE * 16
_DUMMY_ROW = _N
_DEGW = 128


def _lane_cumsum(x):
    r, c = x.shape
    for s in (1, 2, 4, 8, 16, 32, 64):
        shifted = jnp.concatenate(
            [jnp.zeros((r, s), x.dtype), x[:, : c - s]], axis=1)
        x = x + shifted
    return x


def _maxk(h):
    bits = lax.bitcast_convert_type(h, jnp.uint32)
    sign = bits >> jnp.uint32(31)
    key = bits ^ ((sign * jnp.uint32(0x7FFFFFFF)) | jnp.uint32(0x80000000))

    t = jnp.zeros((h.shape[0], 1), jnp.uint32)

    def body(i, t):
        b = jnp.uint32(31) - i.astype(jnp.uint32)
        cand = t | (jnp.uint32(1) << b)
        cnt = jnp.sum((key >= cand).astype(jnp.int32), axis=1, keepdims=True)
        return jnp.where(cnt >= _K, cand, t)

    t = lax.fori_loop(0, 32, body, t)

    gt = key > t
    eq = key == t
    c_gt = jnp.sum(gt.astype(jnp.int32), axis=1, keepdims=True)
    need = _K - c_gt
    rank = _lane_cumsum(eq.astype(jnp.int32))
    keep = gt | (eq & (rank <= need))
    return h * keep.astype(h.dtype)


def _front_body(x_ref, wint_ref, bin_ref, wself_ref, bself_ref,
                h_ref, s_ref):
    h = jnp.dot(x_ref[...], wint_ref[...],
                preferred_element_type=jnp.float32) + bin_ref[...]
    h = _maxk(h)
    h_ref[...] = h
    s_ref[...] = jnp.dot(h, wself_ref[...],
                         preferred_element_type=jnp.float32) + bself_ref[...]


def _neigh(s, aggp_ref, degp_ref, wneigh_ref):
    agg = aggp_ref[0] + aggp_ref[1]
    deg = degp_ref[0, :, 0:1] + degp_ref[1, :, 0:1]
    hn = agg / jnp.maximum(deg, 1.0)
    return s + jnp.dot(hn, wneigh_ref[...], preferred_element_type=jnp.float32)


def _mid_body(s_ref, aggp_ref, degp_ref, wneigh_ref, wself_ref, bself_ref,
              h_out_ref, s_out_ref):
    h = _maxk(_neigh(s_ref[...], aggp_ref, degp_ref, wneigh_ref))
    h_out_ref[...] = h
    s_out_ref[...] = jnp.dot(h, wself_ref[...],
                             preferred_element_type=jnp.float32) + bself_ref[...]


def _back_body(s_ref, aggp_ref, degp_ref, wneigh_ref, wout_ref, bout_ref,
               out_ref):
    h = _neigh(s_ref[...], aggp_ref, degp_ref, wneigh_ref)
    out_ref[...] = jnp.dot(h, wout_ref[...],
                           preferred_element_type=jnp.float32) + bout_ref[...]


_f32 = jnp.float32
_NPF = jax.ShapeDtypeStruct((_N, _D), _f32)
_BLK = 2000
_GRID = _N // _BLK

_rows = pl.BlockSpec((_BLK, _D), lambda i: (i, 0))
_w128 = pl.BlockSpec((_D, _D), lambda i: (0, 0))
_b128 = pl.BlockSpec((1, _D), lambda i: (0, 0))
_prow = pl.BlockSpec((2, _BLK, _D), lambda i: (0, i, 0))
_pdeg = pl.BlockSpec((2, _BLK, _DEGW), lambda i: (0, i, 0))

_front_call = pl.pallas_call(
    _front_body, out_shape=(_NPF, _NPF), grid=(_GRID,),
    in_specs=[_rows, _w128, _b128, _w128, _b128],
    out_specs=(_rows, _rows))
_mid_call = pl.pallas_call(
    _mid_body, out_shape=(_NPF, _NPF), grid=(_GRID,),
    in_specs=[_rows, _prow, _pdeg, _w128, _w128, _b128],
    out_specs=(_rows, _rows))
_back_call = pl.pallas_call(
    _back_body, out_shape=_NPF, grid=(_GRID,),
    in_specs=[_rows, _prow, _pdeg, _w128, _w128, _b128],
    out_specs=_rows)


@functools.lru_cache(maxsize=None)
def _sc_agg_kernel():
    mesh = plsc.VectorSubcoreMesh(core_axis_name="c", subcore_axis_name="s")

    scratch = [
        pltpu.VMEM_SHARED((_NPAD, _D), _f32),
        pltpu.VMEM((_AQNB, _AEB), jnp.int32),
        pltpu.VMEM((_AQNB, _AEB), jnp.int32),
        pltpu.VMEM((_NBUF, _AEB, _D), _f32),
    ] + [pltpu.SemaphoreType.DMA] * _NBUF

    @functools.partial(pl.kernel,
                       out_type=jax.ShapeDtypeStruct((2, _NPAD, _D), _f32),
                       mesh=mesh, scratch_types=scratch)
    def k(z_hbm, src_hbm, dst_hbm, zrow_hbm, agg_out,
          acc, src_v, dst_v, gbuf, *sems):
        cid = lax.axis_index("c")
        sid = lax.axis_index("s")
        wid = sid * 2 + cid
        base = sid * _ROWS_PER_TILE

        pltpu.sync_copy(zrow_hbm, acc.at[pl.ds(base, _ROWS_PER_TILE)])
        plsc.subcore_barrier()

        for q in range(_AQ):
            pltpu.sync_copy(src_hbm.at[wid, pl.ds(q * _AQNB, _AQNB)], src_v)
            pltpu.sync_copy(dst_hbm.at[wid, pl.ds(q * _AQNB, _AQNB)], dst_v)
            for b in range(_NBUF):
                pltpu.async_copy(z_hbm.at[src_v.at[b]], gbuf.at[b], sems[b])

            def body(i, carry):
                j = i * _NBUF
                for b in range(_NBUF):
                    pltpu.make_async_copy(
                        z_hbm.at[src_v.at[j + b]], gbuf.at[b], sems[b]).wait()
                    pltpu.sync_copy(gbuf.at[b], acc.at[dst_v.at[j + b]],
                                    add=True)
                    pltpu.async_copy(
                        z_hbm.at[src_v.at[j + b + _NBUF]], gbuf.at[b],
                        sems[b])
                return carry

            lax.fori_loop(0, (_AQNB - _NBUF) // _NBUF, body, 0)

            for b in range(_NBUF):
                pltpu.make_async_copy(
                    z_hbm.at[src_v.at[_AQNB - _NBUF + b]], gbuf.at[b],
                    sems[b]).wait()
                pltpu.sync_copy(
                    gbuf.at[b], acc.at[dst_v.at[_AQNB - _NBUF + b]],
                    add=True)
        plsc.subcore_barrier()

        pltpu.sync_copy(acc.at[pl.ds(base, _ROWS_PER_TILE)],
                        agg_out.at[cid, pl.ds(base, _ROWS_PER_TILE)])

    return k


@functools.lru_cache(maxsize=None)
def _sc_deg_kernel():
    mesh = plsc.VectorSubcoreMesh(core_axis_name="c", subcore_axis_name="s")

    scratch = [
        pltpu.VMEM_SHARED((_NPAD, _DEGW), _f32),
        pltpu.VMEM((_DNB, _DEB), jnp.int32),
        pltpu.VMEM((_DEB, _DEGW), _f32),
    ]

    @functools.partial(pl.kernel,
                       out_type=jax.ShapeDtypeStruct((2, _NPAD, _DEGW), _f32),
                       mesh=mesh, scratch_types=scratch)
    def k(dst_hbm, zdeg_hbm, ones_hbm, deg_out, degacc, dst_v, ones_v):
        cid = lax.axis_index("c")
        sid = lax.axis_index("s")
        wid = sid * 2 + cid
        base = sid * _ROWS_PER_TILE

        pltpu.sync_copy(zdeg_hbm, degacc.at[pl.ds(base, _ROWS_PER_TILE)])
        pltpu.sync_copy(ones_hbm, ones_v)
        pltpu.sync_copy(dst_hbm.at[wid], dst_v)
        plsc.subcore_barrier()

        def body(j, carry):
            pltpu.sync_copy(ones_v, degacc.at[dst_v.at[j]], add=True)
            return carry

        lax.fori_loop(0, _DNB, body, 0)
        plsc.subcore_barrier()

        pltpu.sync_copy(degacc.at[pl.ds(base, _ROWS_PER_TILE)],
                        deg_out.at[cid, pl.ds(base, _ROWS_PER_TILE)])

    return k


def _pad_edges(idx, fill):
    w = idx.reshape(_NW, _N)
    pad = jnp.full((_NW, _EPW - _N), fill, jnp.int32)
    return jnp.concatenate([w, pad], axis=1)


def kernel(x, edge_index, W_in, b_in, W_self0, b_self0, W_neigh0,
           W_self1, b_self1, W_neigh1, W_out, b_out):
    src_flat = _pad_edges(edge_index[0], 0)
    dst_flat = _pad_edges(edge_index[1], _DUMMY_ROW)
    src3 = src_flat.reshape(_NW, _ANB, _AEB)
    dst3 = dst_flat.reshape(_NW, _ANB, _AEB)
    dst3d = dst_flat.reshape(_NW, _DNB, _DEB)

    zrow = jnp.zeros((_ROWS_PER_TILE, _D), _f32)
    ones = jnp.ones((_DEB, _DEGW), _f32)

    b_in2 = b_in.reshape(1, _D)
    b_self02 = b_self0.reshape(1, _D)
    b_self12 = b_self1.reshape(1, _D)
    b_out2 = b_out.reshape(1, _D)

    h0, s0 = _front_call(x, W_in.T, b_in2, W_self0.T, b_self02)
    degp0 = _sc_deg_kernel()(dst3d, zrow, ones)
    aggp0 = _sc_agg_kernel()(h0, src3, dst3, zrow)
    h1, s1 = _mid_call(s0, aggp0, degp0, W_neigh0.T, W_self1.T, b_self12)
    aggp1 = _sc_agg_kernel()(h1, src3, dst3, zrow)
    return _back_call(s1, aggp1, degp0, W_neigh1.T, W_out.T, b_out2)


# speedup vs baseline: 1.0894x; 1.0894x over previous
import functools

import jax
import jax.numpy as jnp
from jax import lax
from jax.experimental import pallas as pl
from jax.experimental.pallas import tpu as pltpu
from jax.experimental.pallas import tpu_sc as plsc

_N = 10000
_D = 128
_K = 32

_NW = 32
_EB = 64
_EPW = 10112
_NB = _EPW // _EB
_ROWS_PER_TILE = 632
_NPAD = _ROWS_PER_TILE * 16
_DUMMY_ROW = _N
_DEGW = 128


def _lane_cumsum(x):
    r, c = x.shape
    for s in (1, 2, 4, 8, 16, 32, 64):
        shifted = jnp.concatenate(
            [jnp.zeros((r, s), x.dtype), x[:, : c - s]], axis=1)
        x = x + shifted
    return x


def _maxk(h):
    bits = lax.bitcast_convert_type(h, jnp.uint32)
    sign = bits >> jnp.uint32(31)
    key = bits ^ ((sign * jnp.uint32(0x7FFFFFFF)) | jnp.uint32(0x80000000))

    t = jnp.zeros((h.shape[0], 1), jnp.uint32)

    def body(i, t):
        b = jnp.uint32(31) - i.astype(jnp.uint32)
        cand = t | (jnp.uint32(1) << b)
        cnt = jnp.sum((key >= cand).astype(jnp.int32), axis=1, keepdims=True)
        return jnp.where(cnt >= _K, cand, t)

    t = lax.fori_loop(0, 32, body, t)

    gt = key > t
    eq = key == t
    c_gt = jnp.sum(gt.astype(jnp.int32), axis=1, keepdims=True)
    need = _K - c_gt
    rank = _lane_cumsum(eq.astype(jnp.int32))
    keep = gt | (eq & (rank <= need))
    return h * keep.astype(h.dtype)


def _front_body(x_ref, wint_ref, bin_ref, wself_ref, bself_ref,
                h_ref, s_ref):
    h = jnp.dot(x_ref[...], wint_ref[...],
                preferred_element_type=jnp.float32) + bin_ref[...]
    h = _maxk(h)
    h_ref[...] = h
    s_ref[...] = jnp.dot(h, wself_ref[...],
                         preferred_element_type=jnp.float32) + bself_ref[...]


def _neigh(s, aggp_ref, degp_ref, wneigh_ref):
    agg = aggp_ref[0] + aggp_ref[1]
    deg = degp_ref[0, :, 0:1] + degp_ref[1, :, 0:1]
    hn = agg / jnp.maximum(deg, 1.0)
    return s + jnp.dot(hn, wneigh_ref[...], preferred_element_type=jnp.float32)


def _mid_body(s_ref, aggp_ref, degp_ref, wneigh_ref, wself_ref, bself_ref,
              h_out_ref, s_out_ref):
    h = _maxk(_neigh(s_ref[...], aggp_ref, degp_ref, wneigh_ref))
    h_out_ref[...] = h
    s_out_ref[...] = jnp.dot(h, wself_ref[...],
                             preferred_element_type=jnp.float32) + bself_ref[...]


def _back_body(s_ref, aggp_ref, degp_ref, wneigh_ref, wout_ref, bout_ref,
               out_ref):
    h = _neigh(s_ref[...], aggp_ref, degp_ref, wneigh_ref)
    out_ref[...] = jnp.dot(h, wout_ref[...],
                           preferred_element_type=jnp.float32) + bout_ref[...]


_f32 = jnp.float32
_NPF = jax.ShapeDtypeStruct((_N, _D), _f32)
_BLK = 2000
_GRID = _N // _BLK

_rows = pl.BlockSpec((_BLK, _D), lambda i: (i, 0))
_w128 = pl.BlockSpec((_D, _D), lambda i: (0, 0))
_b128 = pl.BlockSpec((1, _D), lambda i: (0, 0))
_prow = pl.BlockSpec((2, _BLK, _D), lambda i: (0, i, 0))
_pdeg = pl.BlockSpec((2, _BLK, _DEGW), lambda i: (0, i, 0))

_front_call = pl.pallas_call(
    _front_body, out_shape=(_NPF, _NPF), grid=(_GRID,),
    in_specs=[_rows, _w128, _b128, _w128, _b128],
    out_specs=(_rows, _rows))
_mid_call = pl.pallas_call(
    _mid_body, out_shape=(_NPF, _NPF), grid=(_GRID,),
    in_specs=[_rows, _prow, _pdeg, _w128, _w128, _b128],
    out_specs=(_rows, _rows))
_back_call = pl.pallas_call(
    _back_body, out_shape=_NPF, grid=(_GRID,),
    in_specs=[_rows, _prow, _pdeg, _w128, _w128, _b128],
    out_specs=_rows)


@functools.lru_cache(maxsize=None)
def _sc_agg_kernel():
    mesh = plsc.VectorSubcoreMesh(core_axis_name="c", subcore_axis_name="s")

    scratch = [
        pltpu.VMEM_SHARED((_NPAD, _D), _f32),
        pltpu.VMEM((_NB, _EB), jnp.int32),
        pltpu.VMEM((_NB, _EB), jnp.int32),
        pltpu.VMEM((_EB, _D), _f32),
        pltpu.SemaphoreType.DMA,
    ]

    @functools.partial(pl.kernel,
                       out_type=jax.ShapeDtypeStruct((2, _NPAD, _D), _f32),
                       mesh=mesh, scratch_types=scratch)
    def k(z_hbm, src_hbm, dst_hbm, zrow_hbm, agg_out,
          acc, src_v, dst_v, gbuf, sem):
        cid = lax.axis_index("c")
        sid = lax.axis_index("s")
        wid = sid * 2 + cid
        base = sid * _ROWS_PER_TILE

        pltpu.sync_copy(zrow_hbm, acc.at[pl.ds(base, _ROWS_PER_TILE)])
        pltpu.sync_copy(src_hbm.at[wid], src_v)
        pltpu.sync_copy(dst_hbm.at[wid], dst_v)
        plsc.subcore_barrier()

        def body(j, carry):
            pltpu.async_copy(z_hbm.at[src_v.at[j]], gbuf, sem).wait()
            pltpu.sync_copy(gbuf, acc.at[dst_v.at[j]], add=True)
            return carry

        lax.fori_loop(0, _NB, body, 0)
        plsc.subcore_barrier()

        pltpu.sync_copy(acc.at[pl.ds(base, _ROWS_PER_TILE)],
                        agg_out.at[cid, pl.ds(base, _ROWS_PER_TILE)])

    return k


@functools.lru_cache(maxsize=None)
def _sc_deg_kernel():
    mesh = plsc.VectorSubcoreMesh(core_axis_name="c", subcore_axis_name="s")

    scratch = [
        pltpu.VMEM_SHARED((_NPAD, _DEGW), _f32),
        pltpu.VMEM((_NB, _EB), jnp.int32),
        pltpu.VMEM((_EB, _DEGW), _f32),
    ]

    @functools.partial(pl.kernel,
                       out_type=jax.ShapeDtypeStruct((2, _NPAD, _DEGW), _f32),
                       mesh=mesh, scratch_types=scratch)
    def k(dst_hbm, zdeg_hbm, ones_hbm, deg_out, degacc, dst_v, ones_v):
        cid = lax.axis_index("c")
        sid = lax.axis_index("s")
        wid = sid * 2 + cid
        base = sid * _ROWS_PER_TILE

        pltpu.sync_copy(zdeg_hbm, degacc.at[pl.ds(base, _ROWS_PER_TILE)])
        pltpu.sync_copy(ones_hbm, ones_v)
        pltpu.sync_copy(dst_hbm.at[wid], dst_v)
        plsc.subcore_barrier()

        def body(j, carry):
            pltpu.sync_copy(ones_v, degacc.at[dst_v.at[j]], add=True)
            return carry

        lax.fori_loop(0, _NB, body, 0)
        plsc.subcore_barrier()

        pltpu.sync_copy(degacc.at[pl.ds(base, _ROWS_PER_TILE)],
                        deg_out.at[cid, pl.ds(base, _ROWS_PER_TILE)])

    return k


def _pad_edges(idx, fill):
    w = idx.reshape(_NW, _N)
    pad = jnp.full((_NW, _EPW - _N), fill, jnp.int32)
    return jnp.concatenate([w, pad], axis=1).reshape(_NW, _NB, _EB)


def kernel(x, edge_index, W_in, b_in, W_self0, b_self0, W_neigh0,
           W_self1, b_self1, W_neigh1, W_out, b_out):
    src3 = _pad_edges(edge_index[0], 0)
    dst3 = _pad_edges(edge_index[1], _DUMMY_ROW)

    zrow = jnp.zeros((_ROWS_PER_TILE, _D), _f32)
    ones = jnp.ones((_EB, _DEGW), _f32)

    b_in2 = b_in.reshape(1, _D)
    b_self02 = b_self0.reshape(1, _D)
    b_self12 = b_self1.reshape(1, _D)
    b_out2 = b_out.reshape(1, _D)

    h0, s0 = _front_call(x, W_in.T, b_in2, W_self0.T, b_self02)
    degp0 = _sc_deg_kernel()(dst3, zrow, ones)
    aggp0 = _sc_agg_kernel()(h0, src3, dst3, zrow)
    h1, s1 = _mid_call(s0, aggp0, degp0, W_neigh0.T, W_self1.T, b_self12)
    aggp1 = _sc_agg_kernel()(h1, src3, dst3, zrow)
    return _back_call(s1, aggp1, degp0, W_neigh1.T, W_out.T, b_out2)
